# baseline (speedup 1.0000x reference)
import functools

import jax
import jax.numpy as jnp
from jax import lax
from jax.experimental import pallas as pl
from jax.experimental.pallas import tpu as pltpu
from jax.experimental.pallas import tpu_sc as plsc

_B = 4
_T = 2048
_D = 768
_NC = 2
_NS = 16
_NW = _NC * _NS
_PPW = _T // _NW
_CH = 32
_SUB = _PPW // _CH
_NCH = _B * _SUB
_NV = _D // 16
_NBUF = 3
_DEPTH = 2


def _emb_kernel(idx_hbm, tok_hbm, pos_hbm, out_hbm,
                idx_v, pos_v, bufs, isem, psem, gsems, osems):
    wid = lax.axis_index("s") * _NC + lax.axis_index("c")
    pos_base = wid * _PPW

    pos_d = pltpu.async_copy(pos_hbm.at[pl.ds(pos_base, _PPW)], pos_v, psem)
    idx_d = [
        pltpu.async_copy(
            idx_hbm.at[b, pl.ds(pos_base, _PPW)], idx_v.at[b], isem
        )
        for b in range(_B)
    ]

    def add_rows(buf, s):
        @plsc.parallel_loop(0, _CH, 1, unroll=4)
        def row_body(j):
            p = s * _CH + j
            for k in range(_NV):
                col = k * 16
                plsc.addupdate(
                    buf.at[j, pl.ds(col, 16)], pos_v[p, pl.ds(col, 16)]
                )

    def start_gather(c):
        b, s = c // _SUB, c % _SUB
        if s == 0:
            idx_d[b].wait()
        return pltpu.async_copy(
            tok_hbm.at[idx_v.at[b, pl.ds(s * _CH, _CH)]],
            bufs[c % _NBUF],
            gsems[c % _NBUF],
        )

    gd = [None] * _NCH
    od = [None] * _NCH
    for c in range(_DEPTH):
        gd[c] = start_gather(c)
    pos_d.wait()
    for c in range(_NCH):
        p = c % _NBUF
        b, s = c // _SUB, c % _SUB
        gd[c].wait()
        if c + _DEPTH < _NCH:
            prev = c + _DEPTH - _NBUF
            if prev >= 0:
                od[prev].wait()
            gd[c + _DEPTH] = start_gather(c + _DEPTH)
        add_rows(bufs[p], s)
        od[c] = pltpu.async_copy(
            bufs[p],
            out_hbm.at[b, pl.ds(pos_base + s * _CH, _CH)],
            osems[p],
        )
    for c in range(max(0, _NCH - _NBUF), _NCH):
        od[c].wait()


def kernel(idx, tok_weight, pos_weight):
    idx32 = idx.astype(jnp.int32)
    mesh = plsc.VectorSubcoreMesh(core_axis_name="c", subcore_axis_name="s")
    run = functools.partial(
        pl.kernel,
        out_type=jax.ShapeDtypeStruct((_B, _T, _D), jnp.float32),
        mesh=mesh,
        scratch_types=[
            pltpu.VMEM((_B, _PPW), jnp.int32),
            pltpu.VMEM((_PPW, _D), jnp.float32),
            [pltpu.VMEM((_CH, _D), jnp.float32) for _ in range(_NBUF)],
            pltpu.SemaphoreType.DMA,
            pltpu.SemaphoreType.DMA,
            [pltpu.SemaphoreType.DMA for _ in range(_NBUF)],
            [pltpu.SemaphoreType.DMA for _ in range(_NBUF)],
        ],
    )(_emb_kernel)
    return run(idx32, tok_weight, pos_weight)


# speedup vs baseline: 1.1606x; 1.1606x over previous
import functools

import jax
import jax.numpy as jnp
from jax import lax
from jax.experimental import pallas as pl
from jax.experimental.pallas import tpu as pltpu
from jax.experimental.pallas import tpu_sc as plsc

_B = 4
_T = 2048
_D = 768
_NC = 2
_NS = 16
_NW = _NC * _NS
_PPW = _T // _NW
_CH = 16
_NG = _PPW // _CH
_NV = _D // 16


def _emb_kernel(idx_hbm, tok_hbm, pos_hbm, out_hbm,
                idx_v, pos_bufs, bufs, isem, psems, gsems, osems):
    wid = lax.axis_index("s") * _NC + lax.axis_index("c")
    pos_base = wid * _PPW

    idx_d = [
        pltpu.async_copy(
            idx_hbm.at[b, pl.ds(pos_base, _PPW)], idx_v.at[b], isem
        )
        for b in range(_B)
    ]

    def start_pos(s):
        return pltpu.async_copy(
            pos_hbm.at[pl.ds(pos_base + s * _CH, _CH)],
            pos_bufs[s % 2],
            psems[s % 2],
        )

    def start_gather(s, b):
        return pltpu.async_copy(
            tok_hbm.at[idx_v.at[b, pl.ds(s * _CH, _CH)]],
            bufs[(s % 2) * _B + b],
            gsems[(s % 2) * _B + b],
        )

    def add_group(s):
        slot = (s % 2) * _B
        pbuf = pos_bufs[s % 2]

        @plsc.parallel_loop(0, _CH, 1, unroll=2)
        def row_body(j):
            for k in range(_NV):
                col = k * 16
                v = pbuf[j, pl.ds(col, 16)]
                for b in range(_B):
                    plsc.addupdate(bufs[slot + b].at[j, pl.ds(col, 16)], v)

    pos_d = [None] * _NG
    gd = [[None] * _B for _ in range(_NG)]
    od = [[None] * _B for _ in range(_NG)]

    pos_d[0] = start_pos(0)
    pos_d[1] = start_pos(1)
    for b in range(_B):
        idx_d[b].wait()
    for s in range(2):
        for b in range(_B):
            gd[s][b] = start_gather(s, b)

    for s in range(_NG):
        pos_d[s].wait()
        for b in range(_B):
            gd[s][b].wait()
        add_group(s)
        for b in range(_B):
            od[s][b] = pltpu.async_copy(
                bufs[(s % 2) * _B + b],
                out_hbm.at[b, pl.ds(pos_base + s * _CH, _CH)],
                osems[(s % 2) * _B + b],
            )
        if s + 2 < _NG:
            pos_d[s + 2] = start_pos(s + 2)
            for b in range(_B):
                od[s][b].wait()
                gd[s + 2][b] = start_gather(s + 2, b)

    for s in range(_NG - 2, _NG):
        for b in range(_B):
            od[s][b].wait()


def kernel(idx, tok_weight, pos_weight):
    idx32 = idx.astype(jnp.int32)
    mesh = plsc.VectorSubcoreMesh(core_axis_name="c", subcore_axis_name="s")
    run = functools.partial(
        pl.kernel,
        out_type=jax.ShapeDtypeStruct((_B, _T, _D), jnp.float32),
        mesh=mesh,
        scratch_types=[
            pltpu.VMEM((_B, _PPW), jnp.int32),
            [pltpu.VMEM((_CH, _D), jnp.float32) for _ in range(2)],
            [pltpu.VMEM((_CH, _D), jnp.float32) for _ in range(2 * _B)],
            pltpu.SemaphoreType.DMA,
            [pltpu.SemaphoreType.DMA for _ in range(2)],
            [pltpu.SemaphoreType.DMA for _ in range(2 * _B)],
            [pltpu.SemaphoreType.DMA for _ in range(2 * _B)],
        ],
    )(_emb_kernel)
    return run(idx32, tok_weight, pos_weight)


# speedup vs baseline: 1.2414x; 1.0696x over previous
import functools

import jax
import jax.numpy as jnp
from jax import lax
from jax.experimental import pallas as pl
from jax.experimental.pallas import tpu as pltpu
from jax.experimental.pallas import tpu_sc as plsc

_B = 4
_T = 2048
_D = 768
_NC = 2
_NS = 16
_NW = _NC * _NS
_PPW = _T // _NW
_CH = 16
_NG = _PPW // _CH
_NV = _D // 16


def _emb_kernel(idx_hbm, tok_hbm, pos_hbm, out_hbm,
                idx_v, pos_bufs, bufs, isem, psems, gsems, osems):
    wid = lax.axis_index("s") * _NC + lax.axis_index("c")
    pos_base = wid * _PPW

    idx_d = [
        pltpu.async_copy(
            idx_hbm.at[b, pl.ds(pos_base, _PPW)], idx_v.at[b], isem
        )
        for b in range(_B)
    ]

    def start_pos(s):
        return pltpu.async_copy(
            pos_hbm.at[pl.ds(pos_base + s * _CH, _CH)],
            pos_bufs[s % 2],
            psems[s % 2],
        )

    def start_gather(s, b):
        return pltpu.async_copy(
            tok_hbm.at[idx_v.at[b, pl.ds(s * _CH, _CH)]],
            bufs[(s % 2) * _B + b],
            gsems[(s % 2) * _B + b],
        )

    def add_group(s):
        slot = (s % 2) * _B
        pbuf = pos_bufs[s % 2]

        @plsc.parallel_loop(0, _CH, 1, unroll=1)
        def row_body(j):
            for k in range(_NV):
                col = k * 16
                v = pbuf[j, pl.ds(col, 16)]
                for b in range(_B):
                    plsc.addupdate(bufs[slot + b].at[j, pl.ds(col, 16)], v)

    pos_d = [None] * _NG
    gd = [[None] * _B for _ in range(_NG)]
    od = [[None] * _B for _ in range(_NG)]

    pos_d[0] = start_pos(0)
    pos_d[1] = start_pos(1)
    for b in range(_B):
        idx_d[b].wait()
    for s in range(2):
        for b in range(_B):
            gd[s][b] = start_gather(s, b)

    for s in range(_NG):
        pos_d[s].wait()
        for b in range(_B):
            gd[s][b].wait()
        add_group(s)
        for b in range(_B):
            od[s][b] = pltpu.async_copy(
                bufs[(s % 2) * _B + b],
                out_hbm.at[b, pl.ds(pos_base + s * _CH, _CH)],
                osems[(s % 2) * _B + b],
            )
        if s + 2 < _NG:
            pos_d[s + 2] = start_pos(s + 2)
            for b in range(_B):
                od[s][b].wait()
                gd[s + 2][b] = start_gather(s + 2, b)

    for s in range(_NG - 2, _NG):
        for b in range(_B):
            od[s][b].wait()


def kernel(idx, tok_weight, pos_weight):
    idx32 = idx.astype(jnp.int32)
    mesh = plsc.VectorSubcoreMesh(core_axis_name="c", subcore_axis_name="s")
    run = functools.partial(
        pl.kernel,
        out_type=jax.ShapeDtypeStruct((_B, _T, _D), jnp.float32),
        mesh=mesh,
        scratch_types=[
            pltpu.VMEM((_B, _PPW), jnp.int32),
            [pltpu.VMEM((_CH, _D), jnp.float32) for _ in range(2)],
            [pltpu.VMEM((_CH, _D), jnp.float32) for _ in range(2 * _B)],
            pltpu.SemaphoreType.DMA,
            [pltpu.SemaphoreType.DMA for _ in range(2)],
            [pltpu.SemaphoreType.DMA for _ in range(2 * _B)],
            [pltpu.SemaphoreType.DMA for _ in range(2 * _B)],
        ],
    )(_emb_kernel)
    return run(idx32, tok_weight, pos_weight)


# speedup vs baseline: 1.3122x; 1.0571x over previous
import functools

import jax
import jax.numpy as jnp
from jax import lax
from jax.experimental import pallas as pl
from jax.experimental.pallas import tpu as pltpu
from jax.experimental.pallas import tpu_sc as plsc

_B = 4
_T = 2048
_D = 768
_NC = 2
_NS = 16
_NW = _NC * _NS
_PPW = _T // _NW
_CH = 16
_NG = _PPW // _CH
_NV = _D // 16


def _emb_kernel(idx_hbm, tok_hbm, pos_hbm, out_hbm,
                idx_v, pos_bufs, bufs, isem, psems, gsems, osems):
    wid = lax.axis_index("s") * _NC + lax.axis_index("c")
    pos_base = wid * _PPW

    idx_d = [
        pltpu.async_copy(
            idx_hbm.at[b, pl.ds(pos_base, _PPW)], idx_v.at[b], isem
        )
        for b in range(_B)
    ]

    def start_pos(s, u):
        return pltpu.async_copy(
            pos_hbm.at[pl.ds(pos_base + s * _CH, _CH)],
            pos_bufs[u],
            psems[u],
        )

    def start_gather(s, u, b):
        off = pl.multiple_of(s * _CH, 8)
        return pltpu.async_copy(
            tok_hbm.at[idx_v.at[b, pl.ds(off, _CH)]],
            bufs[u * _B + b],
            gsems[u * _B + b],
        )

    def add_group(u):
        pbuf = pos_bufs[u]

        @plsc.parallel_loop(0, _CH, 1, unroll=1)
        def row_body(j):
            for k in range(_NV):
                col = k * 16
                v = pbuf[j, pl.ds(col, 16)]
                for b in range(_B):
                    plsc.addupdate(
                        bufs[u * _B + b].at[j, pl.ds(col, 16)], v
                    )

    for u in range(2):
        start_pos(u, u)
    for b in range(_B):
        idx_d[b].wait()
    for u in range(2):
        for b in range(_B):
            start_gather(u, u, b)

    def pair_body(t, _):
        for u in range(2):
            s = 2 * t + u
            pltpu.make_async_copy(
                pos_hbm.at[pl.ds(pos_base, _CH)], pos_bufs[u], psems[u]
            ).wait()
            for b in range(_B):
                pltpu.make_async_copy(
                    tok_hbm.at[idx_v.at[b, pl.ds(0, _CH)]],
                    bufs[u * _B + b],
                    gsems[u * _B + b],
                ).wait()
            add_group(u)
            for b in range(_B):
                pltpu.async_copy(
                    bufs[u * _B + b],
                    out_hbm.at[b, pl.ds(pos_base + s * _CH, _CH)],
                    osems[u * _B + b],
                )

            @pl.when(t == 0)
            def _prefetch():
                start_pos(s + 2, u)
                for b in range(_B):
                    pltpu.make_async_copy(
                        bufs[u * _B + b],
                        out_hbm.at[b, pl.ds(pos_base, _CH)],
                        osems[u * _B + b],
                    ).wait()
                    start_gather(s + 2, u, b)

        return ()

    lax.fori_loop(0, _NG // 2, pair_body, ())

    for u in range(2):
        for b in range(_B):
            pltpu.make_async_copy(
                bufs[u * _B + b],
                out_hbm.at[b, pl.ds(pos_base, _CH)],
                osems[u * _B + b],
            ).wait()


def kernel(idx, tok_weight, pos_weight):
    idx32 = idx.astype(jnp.int32)
    mesh = plsc.VectorSubcoreMesh(core_axis_name="c", subcore_axis_name="s")
    run = functools.partial(
        pl.kernel,
        out_type=jax.ShapeDtypeStruct((_B, _T, _D), jnp.float32),
        mesh=mesh,
        scratch_types=[
            pltpu.VMEM((_B, _PPW), jnp.int32),
            [pltpu.VMEM((_CH, _D), jnp.float32) for _ in range(2)],
            [pltpu.VMEM((_CH, _D), jnp.float32) for _ in range(2 * _B)],
            pltpu.SemaphoreType.DMA,
            [pltpu.SemaphoreType.DMA for _ in range(2)],
            [pltpu.SemaphoreType.DMA for _ in range(2 * _B)],
            [pltpu.SemaphoreType.DMA for _ in range(2 * _B)],
        ],
    )(_emb_kernel)
    return run(idx32, tok_weight, pos_weight)
